# baseline (speedup 1.0000x reference)
import functools

import jax
import jax.numpy as jnp
from jax import lax
from jax.experimental import pallas as pl
from jax.experimental.pallas import tpu as pltpu
from jax.experimental.pallas import tpu_sc as plsc

_NC = 2
_NS = 16
_NW = _NC * _NS
_LANES = 16
_UNROLL = 4

_B = 4
_S = 4096
_E = 1024
_R = 4
_ROWS_PER_W = _S // _NW
_T = _ROWS_PER_W // _R
_PERIOD = 4


def _sc_body(tok_hbm, pos_hbm, out_hbm,
             pos0, pos1, tok0, tok1, tok2, tok3,
             isem0, isem1, osem0, osem1):
    wid = lax.axis_index("s") * _NC + lax.axis_index("c")
    s0 = wid * _ROWS_PER_W
    pos_bufs = (pos0, pos1)
    tok_sets = (tok0, tok1, tok2, tok3)
    isems = (isem0, isem1)
    osems = (osem0, osem1)

    def row_of(t):
        return s0 + (t % _T) * _R

    def issue_in(t, k):
        row = row_of(t)
        sem = isems[k % 2]
        return [
            pltpu.async_copy(pos_hbm.at[pl.ds(row, _R), :],
                             pos_bufs[k % 2], sem),
            pltpu.async_copy(tok_hbm.at[:, pl.ds(row, _R), :],
                             tok_sets[k % _PERIOD], sem),
        ]

    def wait_in(k):
        pltpu.make_async_copy(pos_hbm.at[pl.ds(s0, _R), :],
                              pos_bufs[k % 2], isems[k % 2]).wait()
        pltpu.make_async_copy(tok_hbm.at[:, pl.ds(s0, _R), :],
                              tok_sets[k % _PERIOD], isems[k % 2]).wait()

    def issue_out(t, k):
        row = row_of(t)
        return [pltpu.async_copy(
            tok_sets[k % _PERIOD],
            out_hbm.at[:, pl.ds(row, _R), :], osems[k % 2])]

    def wait_out(k):
        pltpu.make_async_copy(tok_sets[k % _PERIOD],
                              out_hbm.at[:, pl.ds(s0, _R), :],
                              osems[k % 2]).wait()

    def compute(k):
        pos_v = pos_bufs[k % 2]
        tset = tok_sets[k % _PERIOD]
        for r in range(_R):
            @pl.loop(0, _E, step=_LANES * _UNROLL)
            def _add_loop(c):
                for u in range(_UNROLL):
                    sl = pl.ds(c + u * _LANES, _LANES)
                    p = pos_v[r, sl]
                    for b in range(_B):
                        plsc.addupdate(tset.at[b, r, sl], p)

    in_h = {0: issue_in(0, 0)}
    out_h = {}
    for t in range(_PERIOD):
        if t >= 2:
            for h in out_h[t - 2]:
                h.wait()
        in_h[t + 1] = issue_in(t + 1, (t + 1) % _PERIOD)
        for h in in_h[t]:
            h.wait()
        compute(t % _PERIOD)
        out_h[t] = issue_out(t, t % _PERIOD)

    @pl.loop(1, _T // _PERIOD)
    def _super(j):
        tbase = j * _PERIOD
        for k in range(_PERIOD):
            t = tbase + k
            wait_out((k + 2) % _PERIOD)
            issue_in(t + 1, (k + 1) % _PERIOD)
            wait_in(k)
            compute(k)
            issue_out(t, k)

    wait_out((_T - 2) % _PERIOD)
    wait_out((_T - 1) % _PERIOD)
    wait_in(0)


def kernel(token_embedding, pos_table):
    B, S, E = token_embedding.shape

    mesh = plsc.VectorSubcoreMesh(core_axis_name="c", subcore_axis_name="s")
    run = functools.partial(
        pl.kernel,
        out_type=jax.ShapeDtypeStruct((B, S, E), jnp.float32),
        mesh=mesh,
        scratch_types=(
            [pltpu.VMEM((_R, _E), jnp.float32)] * 2
            + [pltpu.VMEM((_B, _R, _E), jnp.float32)] * _PERIOD
            + [pltpu.SemaphoreType.DMA] * 4
        ),
    )(_sc_body)
    return run(token_embedding, pos_table)


# speedup vs baseline: 1.0650x; 1.0650x over previous
import functools

import jax
import jax.numpy as jnp
from jax import lax
from jax.experimental import pallas as pl
from jax.experimental.pallas import tpu as pltpu
from jax.experimental.pallas import tpu_sc as plsc

_NC = 2
_NS = 16
_NW = _NC * _NS
_LANES = 16
_UNROLL = 4

_B = 4
_S = 4096
_E = 1024
_R = 8
_ROWS_PER_W = _S // _NW
_T = _ROWS_PER_W // _R


def _sc_body(tok_hbm, pos_hbm, out_hbm,
             pos0, pos1, tok0, tok1, tok2,
             isem0, isem1, osem0, osem1):
    wid = lax.axis_index("s") * _NC + lax.axis_index("c")
    s0 = wid * _ROWS_PER_W
    pos_bufs = (pos0, pos1)
    tok_sets = (tok0, tok1, tok2)
    isems = (isem0, isem1)
    osems = (osem0, osem1)

    def issue_in(t):
        row = s0 + t * _R
        sem = isems[t % 2]
        tset = tok_sets[t % 3]
        return [
            pltpu.async_copy(pos_hbm.at[pl.ds(row, _R), :],
                             pos_bufs[t % 2], sem),
            pltpu.async_copy(tok_hbm.at[:, pl.ds(row, _R), :], tset, sem),
        ]

    def issue_out(t):
        row = s0 + t * _R
        tset = tok_sets[t % 3]
        return [pltpu.async_copy(
            tset, out_hbm.at[:, pl.ds(row, _R), :], osems[t % 2])]

    def compute(t):
        pos_v = pos_bufs[t % 2]
        tset = tok_sets[t % 3]

        @pl.loop(0, _R)
        def _row_loop(r):
            @pl.loop(0, _E, step=_LANES * _UNROLL)
            def _add_loop(c):
                for u in range(_UNROLL):
                    sl = pl.ds(c + u * _LANES, _LANES)
                    p = pos_v[r, sl]
                    for b in range(_B):
                        plsc.addupdate(tset.at[b, r, sl], p)

    in_h = {0: issue_in(0)}
    out_h = {}
    for t in range(_T):
        if t >= 2:
            for h in out_h[t - 2]:
                h.wait()
        if t + 1 < _T:
            in_h[t + 1] = issue_in(t + 1)
        for h in in_h[t]:
            h.wait()
        compute(t)
        out_h[t] = issue_out(t)
    for h in out_h[_T - 2]:
        h.wait()
    for h in out_h[_T - 1]:
        h.wait()


def kernel(token_embedding, pos_table):
    B, S, E = token_embedding.shape

    mesh = plsc.VectorSubcoreMesh(core_axis_name="c", subcore_axis_name="s")
    run = functools.partial(
        pl.kernel,
        out_type=jax.ShapeDtypeStruct((B, S, E), jnp.float32),
        mesh=mesh,
        scratch_types=(
            [pltpu.VMEM((_R, _E), jnp.float32)] * 2
            + [pltpu.VMEM((_B, _R, _E), jnp.float32)] * 3
            + [pltpu.SemaphoreType.DMA] * 4
        ),
    )(_sc_body)
    return run(token_embedding, pos_table)
